# Initial kernel scaffold; baseline (speedup 1.0000x reference)
import functools

import numpy as np
import jax
import jax.numpy as jnp
from jax import lax
from jax.experimental import pallas as pl
from jax.experimental.pallas import tpu as pltpu, tpu_sc as plsc

MAXLEN = 200
DIM = 64
BATCH = 4096
SEQ = 200

NC = 2
NS = 16
NW = NC * NS
LANES = 16

B_FLAT = BATCH * SEQ
ROWS_PER_W = B_FLAT // NW
CHUNK = 800
N_CHUNKS = ROWS_PER_W // CHUNK
REPS = CHUNK // SEQ
DREGS = DIM // LANES


def _sinusoidal_pos_emb(maxlen, d_model):
    position = np.arange(maxlen)[:, np.newaxis]
    i = np.arange(d_model)[np.newaxis, :]
    angles = 1.0 / np.power(10000, 2 * (i // 2) / np.float32(d_model))
    angle_rads = position * angles
    angle_rads[:, 0::2] = np.sin(angle_rads[:, 0::2])
    angle_rads[:, 1::2] = np.cos(angle_rads[:, 1::2])
    return angle_rads.astype(np.float32)


_POS_NP = _sinusoidal_pos_emb(MAXLEN, DIM)


def _sc_body(idx_hbm, pos_hbm, table_hbm, out_hbm, idx_v, rows_v, pos_v, sem):
    wid = lax.axis_index("s") * NC + lax.axis_index("c")
    base = wid * ROWS_PER_W

    pltpu.sync_copy(pos_hbm, pos_v)

    def chunk_body(c, _):
        start = base + c * CHUNK
        pltpu.sync_copy(idx_hbm.at[pl.ds(start, CHUNK)], idx_v)
        pltpu.async_copy(table_hbm.at[idx_v], rows_v, sem).wait()

        def row_body(r, _):
            pvals = [pos_v[r, pl.ds(d * LANES, LANES)] for d in range(DREGS)]
            for rep in range(REPS):
                row = rep * SEQ + r
                for d in range(DREGS):
                    sl = pl.ds(d * LANES, LANES)
                    rows_v[row, sl] = rows_v[row, sl] + pvals[d]
            return 0

        lax.fori_loop(0, SEQ, row_body, 0)
        pltpu.sync_copy(rows_v, out_hbm.at[pl.ds(start, CHUNK)])
        return 0

    lax.fori_loop(0, N_CHUNKS, chunk_body, 0)


@jax.jit
def _embed(x_flat, pos, token_table):
    mesh = plsc.VectorSubcoreMesh(core_axis_name="c", subcore_axis_name="s")
    fn = pl.kernel(
        _sc_body,
        out_type=jax.ShapeDtypeStruct((B_FLAT, DIM), jnp.float32),
        mesh=mesh,
        scratch_types=[
            pltpu.VMEM((CHUNK,), jnp.int32),
            pltpu.VMEM((CHUNK, DIM), jnp.float32),
            pltpu.VMEM((SEQ, DIM), jnp.float32),
            pltpu.SemaphoreType.DMA,
        ],
    )
    return fn(x_flat, pos, token_table)


def kernel(x, token_table):
    x_flat = x.reshape(-1).astype(jnp.int32)
    pos = jnp.asarray(_POS_NP)
    out = _embed(x_flat, pos, token_table)
    return out.reshape(BATCH, SEQ, DIM)


# speedup vs baseline: 3.7155x; 3.7155x over previous
import functools

import numpy as np
import jax
import jax.numpy as jnp
from jax import lax
from jax.experimental import pallas as pl
from jax.experimental.pallas import tpu as pltpu, tpu_sc as plsc

MAXLEN = 200
DIM = 64
BATCH = 4096
SEQ = 200

NC = 2
NS = 16
NW = NC * NS
LANES = 16

B_FLAT = BATCH * SEQ
ROWS_PER_W = B_FLAT // NW
CHUNK = 800
N_CHUNKS = ROWS_PER_W // CHUNK
REPS = CHUNK // SEQ
DREGS = DIM // LANES


def _sinusoidal_pos_emb(maxlen, d_model):
    position = np.arange(maxlen)[:, np.newaxis]
    i = np.arange(d_model)[np.newaxis, :]
    angles = 1.0 / np.power(10000, 2 * (i // 2) / np.float32(d_model))
    angle_rads = position * angles
    angle_rads[:, 0::2] = np.sin(angle_rads[:, 0::2])
    angle_rads[:, 1::2] = np.cos(angle_rads[:, 1::2])
    return angle_rads.astype(np.float32)


_POS_NP = _sinusoidal_pos_emb(MAXLEN, DIM)


def _sc_body(idx_hbm, pos_hbm, table_hbm, out_hbm, idx_v, rows_v, pos_v, sem):
    wid = lax.axis_index("s") * NC + lax.axis_index("c")
    base = wid * ROWS_PER_W

    pltpu.sync_copy(pos_hbm, pos_v)

    def chunk_body(c, _):
        start = base + c * CHUNK
        pltpu.sync_copy(idx_hbm.at[pl.ds(start, CHUNK)], idx_v)
        pltpu.async_copy(table_hbm.at[idx_v], rows_v, sem).wait()

        def row_body(r, _):
            pvals = [pos_v[r, pl.ds(d * LANES, LANES)] for d in range(DREGS)]
            for rep in range(REPS):
                row = rep * SEQ + r
                for d in range(DREGS):
                    sl = pl.ds(d * LANES, LANES)
                    rows_v[row, sl] = rows_v[row, sl] + pvals[d]
            return 0

        lax.fori_loop(0, SEQ, row_body, 0)
        pltpu.sync_copy(rows_v, out_hbm.at[pl.ds(start, CHUNK)])
        return 0

    lax.fori_loop(0, N_CHUNKS, chunk_body, 0)


@jax.jit
def _embed(x_flat, token_table):
    pos = jnp.asarray(_POS_NP)
    mesh = plsc.VectorSubcoreMesh(core_axis_name="c", subcore_axis_name="s")
    fn = pl.kernel(
        _sc_body,
        out_type=jax.ShapeDtypeStruct((B_FLAT, DIM), jnp.float32),
        mesh=mesh,
        scratch_types=[
            pltpu.VMEM((CHUNK,), jnp.int32),
            pltpu.VMEM((CHUNK, DIM), jnp.float32),
            pltpu.VMEM((SEQ, DIM), jnp.float32),
            pltpu.SemaphoreType.DMA,
        ],
        compiler_params=pltpu.CompilerParams(use_tc_tiling_on_sc=False),
    )
    return fn(x_flat, pos, token_table)


def kernel(x, token_table):
    x_flat = x.reshape(-1).astype(jnp.int32)
    out = _embed(x_flat, token_table)
    return out.reshape(BATCH, SEQ, DIM)


# speedup vs baseline: 4.2309x; 1.1387x over previous
import functools

import numpy as np
import jax
import jax.numpy as jnp
from jax import lax
from jax.experimental import pallas as pl
from jax.experimental.pallas import tpu as pltpu, tpu_sc as plsc

MAXLEN = 200
DIM = 64
BATCH = 4096
SEQ = 200

NC = 2
NS = 16
NW = NC * NS
LANES = 16

B_FLAT = BATCH * SEQ
ROWS_PER_W = B_FLAT // NW
CHUNK = 800
N_CHUNKS = ROWS_PER_W // CHUNK
REPS = CHUNK // SEQ
DREGS = DIM // LANES


def _sinusoidal_pos_emb(maxlen, d_model):
    position = np.arange(maxlen)[:, np.newaxis]
    i = np.arange(d_model)[np.newaxis, :]
    angles = 1.0 / np.power(10000, 2 * (i // 2) / np.float32(d_model))
    angle_rads = position * angles
    angle_rads[:, 0::2] = np.sin(angle_rads[:, 0::2])
    angle_rads[:, 1::2] = np.cos(angle_rads[:, 1::2])
    return angle_rads.astype(np.float32)


_POS_NP = _sinusoidal_pos_emb(MAXLEN, DIM)


def _sc_body(idx_hbm, pos_hbm, table_hbm, out_hbm,
             idx0, idx1, rows0, rows1, pos_v,
             gsem0, gsem1, wsem0, wsem1):
    idx_b = (idx0, idx1)
    rows_b = (rows0, rows1)
    gsem = (gsem0, gsem1)
    wsem = (wsem0, wsem1)

    wid = lax.axis_index("s") * NC + lax.axis_index("c")
    base = wid * ROWS_PER_W

    pltpu.sync_copy(pos_hbm, pos_v)

    pltpu.sync_copy(idx_hbm.at[pl.ds(base, CHUNK)], idx_b[0])
    pltpu.async_copy(table_hbm.at[idx_b[0]], rows_b[0], gsem[0])

    def pair_body(g, _):
        for b in range(2):
            c = 2 * g + b
            start = base + c * CHUNK

            pltpu.make_async_copy(
                table_hbm.at[idx_b[b]], rows_b[b], gsem[b]).wait()

            @pl.when(c + 1 < N_CHUNKS)
            def _():
                nstart = start + CHUNK
                pltpu.sync_copy(idx_hbm.at[pl.ds(nstart, CHUNK)],
                                idx_b[1 - b])

                @pl.when(c >= 1)
                def _():
                    pltpu.make_async_copy(
                        rows_b[1 - b],
                        out_hbm.at[pl.ds(start - CHUNK, CHUNK)],
                        wsem[1 - b]).wait()

                pltpu.async_copy(table_hbm.at[idx_b[1 - b]],
                                 rows_b[1 - b], gsem[1 - b])

            def row_body(r, _):
                pvals = [pos_v[r, pl.ds(d * LANES, LANES)]
                         for d in range(DREGS)]
                for rep in range(REPS):
                    row = rep * SEQ + r
                    for d in range(DREGS):
                        sl = pl.ds(d * LANES, LANES)
                        rows_b[b][row, sl] = rows_b[b][row, sl] + pvals[d]
                return 0

            lax.fori_loop(0, SEQ, row_body, 0)
            pltpu.async_copy(rows_b[b], out_hbm.at[pl.ds(start, CHUNK)],
                             wsem[b])
        return 0

    lax.fori_loop(0, N_CHUNKS // 2, pair_body, 0)

    end0 = base + (N_CHUNKS - 2) * CHUNK
    end1 = base + (N_CHUNKS - 1) * CHUNK
    pltpu.make_async_copy(rows_b[0], out_hbm.at[pl.ds(end0, CHUNK)],
                          wsem[0]).wait()
    pltpu.make_async_copy(rows_b[1], out_hbm.at[pl.ds(end1, CHUNK)],
                          wsem[1]).wait()


@jax.jit
def _embed(x_flat, token_table):
    pos = jnp.asarray(_POS_NP)
    mesh = plsc.VectorSubcoreMesh(core_axis_name="c", subcore_axis_name="s")
    fn = pl.kernel(
        _sc_body,
        out_type=jax.ShapeDtypeStruct((B_FLAT, DIM), jnp.float32),
        mesh=mesh,
        scratch_types=[
            pltpu.VMEM((CHUNK,), jnp.int32),
            pltpu.VMEM((CHUNK,), jnp.int32),
            pltpu.VMEM((CHUNK, DIM), jnp.float32),
            pltpu.VMEM((CHUNK, DIM), jnp.float32),
            pltpu.VMEM((SEQ, DIM), jnp.float32),
            pltpu.SemaphoreType.DMA,
            pltpu.SemaphoreType.DMA,
            pltpu.SemaphoreType.DMA,
            pltpu.SemaphoreType.DMA,
        ],
        compiler_params=pltpu.CompilerParams(use_tc_tiling_on_sc=False),
    )
    return fn(x_flat, pos, token_table)


def kernel(x, token_table):
    x_flat = x.reshape(-1).astype(jnp.int32)
    out = _embed(x_flat, token_table)
    return out.reshape(BATCH, SEQ, DIM)
